# baseline (speedup 1.0000x reference)
import functools

import jax
import jax.numpy as jnp
from jax import lax
from jax.experimental import pallas as pl
from jax.experimental.pallas import tpu as pltpu
from jax.experimental.pallas import tpu_sc as plsc

_K = 64
_ROWS = 128
_DEPTH = 32768
_NW = 32
_RPW = _ROWS // _NW
_NV = _DEPTH // 16
_CAND_CAP = 8224


def _scal(v):
    return lax.squeeze(lax.slice(v, (0,), (1,)), (0,))


def _sc_stats(x):
    mesh = plsc.VectorSubcoreMesh(
        core_axis_name="c", subcore_axis_name="s", num_cores=2, num_subcores=16)

    @functools.partial(
        pl.kernel,
        out_type=(
            jax.ShapeDtypeStruct((_NW, 16), jnp.int32),
            jax.ShapeDtypeStruct((_NW, 16), jnp.float32),
            jax.ShapeDtypeStruct((_NW, 16), jnp.float32),
        ),
        mesh=mesh,
        compiler_params=pltpu.CompilerParams(needs_layout_passes=False),
        scratch_types=[
            pltpu.VMEM((_DEPTH,), jnp.float32),
            pltpu.VMEM((_DEPTH,), jnp.int32),
            pltpu.VMEM((16 * 2049 + 16,), jnp.int32),
            pltpu.VMEM((2048,), jnp.int32),
            pltpu.VMEM((_CAND_CAP,), jnp.int32),
            pltpu.VMEM((16,), jnp.int32),
            pltpu.VMEM((16,), jnp.float32),
            pltpu.VMEM((16,), jnp.float32),
        ],
    )
    def sc_kernel(x_hbm, t_hbm, xm_hbm, dn_hbm, row_v, ukey_v, priv_v,
                  hist_v, cand_v, t_stage, xm_stage, dn_stage):
        wid = lax.axis_index("s") * 2 + lax.axis_index("c")
        lanes = lax.iota(jnp.int32, 16)
        zeros16 = jnp.zeros((16,), jnp.int32)
        ones16 = jnp.ones((16,), jnp.int32)
        int_min = jnp.int32(-2147483648)
        magn = jnp.int32(0x7FFFFFFF)

        t_acc = zeros16
        xm_acc = jnp.zeros((16,), jnp.float32)
        dn_acc = jnp.zeros((16,), jnp.float32)

        for r in range(_RPW):
            row = wid * _RPW + r
            pltpu.sync_copy(x_hbm.at[row], row_v)

            def clr(i, c):
                priv_v[pl.ds(i * 16, 16)] = zeros16
                return c
            lax.fori_loop(0, 2050, clr, 0, unroll=8)

            laneoff = lanes * 2049

            def h_step(i, c):
                v = row_v[pl.ds(i * 16, 16)]
                bits = lax.bitcast_convert_type(v, jnp.int32)
                m = lax.shift_right_arithmetic(bits, 31)
                ukey = bits ^ (m | int_min)
                ukey_v[pl.ds(i * 16, 16)] = ukey
                bucket = lax.shift_right_logical(ukey, 21)
                plsc.addupdate_scatter(priv_v, [laneoff + bucket], ones16)
                return c
            lax.fori_loop(0, _NV, h_step, 0, unroll=8)

            def mrg(i, c):
                acc = priv_v[pl.ds(i * 16, 16)]
                for l in range(1, 16):
                    acc = acc + priv_v[pl.ds(l * 2049 + i * 16, 16)]
                hist_v[pl.ds(i * 16, 16)] = acc
                return c
            lax.fori_loop(0, 128, mrg, 0, unroll=2)

            def s_step(i, carry):
                acc, jfound, accb = carry
                j = 127 - i
                s = jnp.sum(hist_v[pl.ds(j * 16, 16)])
                hit = jnp.logical_and(acc < _K, acc + s >= _K)
                jfound = jnp.where(hit, j, jfound)
                accb = jnp.where(hit, acc, accb)
                return acc + s, jfound, accb
            _, jc, accc = lax.fori_loop(
                0, 128, s_step, (jnp.int32(0), jnp.int32(0), jnp.int32(0)))

            def _cross(vec, above):
                rev = lax.rev(vec, (0,))
                cum = plsc.cumsum(rev)
                ffs_s = _scal(plsc.all_reduce_ffs((cum + above) >= _K))
                lane = 15 - ffs_s
                c_above = above + jnp.sum(jnp.where(lanes < ffs_s, rev, 0))
                return lane, c_above

            l1, acc1 = _cross(hist_v[pl.ds(jc * 16, 16)], accc)
            del acc1
            bucket_b = jc * 16 + l1
            p_lo_s = lax.shift_left(bucket_b, 21) ^ int_min

            def c_step(i, off):
                ku = ukey_v[pl.ds(i * 16, 16)]
                msk = jnp.logical_and((ku ^ int_min) >= p_lo_s,
                                      off < jnp.int32(_CAND_CAP - 48))
                plsc.store_compressed(cand_v.at[pl.ds(off, 16)], ku, mask=msk)
                cnt = plsc.all_reduce_population_count(msk)
                return off + _scal(cnt)
            ccnt = lax.fori_loop(0, _NV, c_step, jnp.int32(0), unroll=8)

            nv = (ccnt + 15) // 16
            cand_v[pl.ds(ccnt, 16)] = zeros16

            def b_step(i, tpat):
                bit = lax.shift_left(jnp.int32(1), jnp.int32(31) - i)
                cand_s = (tpat | bit) ^ int_min

                def cnt_step(j, accv):
                    kv = cand_v[pl.ds(j * 16, 16)] ^ int_min
                    return accv + plsc.all_reduce_population_count(kv >= cand_s)
                accv = lax.fori_loop(0, nv, cnt_step, zeros16)
                return jnp.where(_scal(accv) >= _K, tpat | bit, tpat)
            tpat = lax.fori_loop(0, 32, b_step, jnp.int32(0))
            t_s = tpat ^ int_min

            def unkey(ku):
                ks = ku ^ int_min
                m2 = lax.shift_right_arithmetic(ks, 31)
                return lax.bitcast_convert_type(ks ^ (m2 & magn), jnp.float32)

            def m_step(j, mv):
                ku = cand_v[pl.ds(j * 16, 16)]
                vv = jnp.where(ku == 0, jnp.float32(-3.4e38), unkey(ku))
                return jnp.maximum(mv, vv)
            mv = lax.fori_loop(
                0, nv, m_step, jnp.full((16,), -3.4e38, jnp.float32))
            xm = jnp.max(mv)

            def d_step(j, carry):
                ev, cv = carry
                ku = cand_v[pl.ds(j * 16, 16)]
                msk = (ku ^ int_min) >= t_s
                ev = ev + jnp.where(msk, jnp.exp(unkey(ku) - xm), 0.0)
                cv = cv + plsc.all_reduce_population_count(msk)
                return ev, cv
            ev, cv = lax.fori_loop(
                0, nv, d_step, (jnp.zeros((16,), jnp.float32), zeros16))
            s_ge = jnp.sum(ev)
            c_ge = _scal(cv)

            bt = jnp.where(t_s < 0, t_s ^ magn, t_s)
            tf = lax.bitcast_convert_type(bt, jnp.float32)
            et = _scal(jnp.exp(jnp.full((16,), tf - xm, jnp.float32)))
            denom = s_ge - (c_ge - _K).astype(jnp.float32) * et

            sel = lanes == r
            t_acc = jnp.where(sel, t_s, t_acc)
            xm_acc = jnp.where(sel, xm, xm_acc)
            dn_acc = jnp.where(sel, denom, dn_acc)

        t_stage[...] = t_acc
        xm_stage[...] = xm_acc
        dn_stage[...] = dn_acc
        pltpu.sync_copy(t_stage, t_hbm.at[wid])
        pltpu.sync_copy(xm_stage, xm_hbm.at[wid])
        pltpu.sync_copy(dn_stage, dn_hbm.at[wid])

    return sc_kernel(x)


def _tc_body(x_ref, t_ref, xm_ref, dn_ref, o_ref):
    mask = jnp.int32(0x7FFFFFFF)
    x = x_ref[...]
    b = lax.bitcast_convert_type(x, jnp.int32)
    keys = jnp.where(b < 0, b ^ mask, b)
    t = t_ref[...]
    e = jnp.exp(x - xm_ref[...])
    o_ref[...] = jnp.where(keys >= t, e / dn_ref[...], 0.0)


@functools.partial(jax.jit, static_argnums=())
def kernel(inputs):
    n_rows, depth = inputs.shape
    t_w, xm_w, dn_w = _sc_stats(inputs)
    t = t_w[:, :_RPW].reshape(n_rows, 1)
    xm = xm_w[:, :_RPW].reshape(n_rows, 1)
    dn = dn_w[:, :_RPW].reshape(n_rows, 1)
    block_rows = 32
    grid = (n_rows // block_rows,)
    small = pl.BlockSpec((block_rows, 1), lambda i: (i, 0))
    return pl.pallas_call(
        _tc_body,
        grid=grid,
        in_specs=[pl.BlockSpec((block_rows, depth), lambda i: (i, 0)),
                  small, small, small],
        out_specs=pl.BlockSpec((block_rows, depth), lambda i: (i, 0)),
        out_shape=jax.ShapeDtypeStruct((n_rows, depth), jnp.float32),
    )(inputs, t, xm, dn)


# speedup vs baseline: 5.7353x; 5.7353x over previous
import functools

import jax
import jax.numpy as jnp
from jax.experimental import pallas as pl

_K = 64


def _count16(pred):
    def rec(lo, hi):
        if hi - lo == 128:
            return jnp.where(pred(lo, hi), jnp.int16(1), jnp.int16(0))
        mid = (lo + hi) // 2
        return rec(lo, mid) + rec(mid, hi)

    m = rec(0, 32768)
    return jnp.sum(m.astype(jnp.int32), axis=1, keepdims=True)


def _as_i16(pat):
    return (pat - 32768).astype(jnp.int16)


def _bisect16(h, rows, extra=None):
    def step(i, t):
        bit = jax.lax.shift_left(jnp.int32(1), jnp.int32(15) - i)
        cand = t | bit
        c16 = _as_i16(cand)
        cnt = _count16(lambda lo, hi: h[:, lo:hi] >= c16)
        if extra is not None:
            cnt = cnt + extra
        return jnp.where(cnt >= _K, cand, t)

    t0 = jnp.zeros((rows, 1), dtype=jnp.int32)
    return jax.lax.fori_loop(0, 16, step, t0, unroll=True)


def _body(x_ref, o_ref):
    mask = jnp.int32(0x7FFFFFFF)
    x = x_ref[...]
    rows = x.shape[0]
    b = jax.lax.bitcast_convert_type(x, jnp.int32)
    keys = jnp.where(b < 0, b ^ mask, b)
    xmax = jnp.max(x, axis=1, keepdims=True)

    hi = jax.lax.shift_right_arithmetic(keys, 16).astype(jnp.int16)
    p_pat = _bisect16(hi, rows)
    p16 = _as_i16(p_pat)

    c_hi_gt = _count16(lambda lo, hi_: hi[:, lo:hi_] > p16)
    lo = _as_i16(keys & jnp.int32(0xFFFF))
    lo_m = jnp.where(hi == p16, lo, jnp.int16(-32768))

    def b_cond(carry):
        j, _, cnt_t = carry
        return jnp.logical_and(j < 16, jnp.logical_not(jnp.all(cnt_t == _K)))

    def b_body(carry):
        j, t, cnt_t = carry
        bit = jax.lax.shift_left(jnp.int32(1), jnp.int32(15) - j)
        cand = t | bit
        c16 = _as_i16(cand)
        cnt = c_hi_gt + _count16(lambda lo_, hi_: lo_m[:, lo_:hi_] >= c16)
        acc = cnt >= _K
        return j + 1, jnp.where(acc, cand, t), jnp.where(acc, cnt, cnt_t)

    _, tl_pat, _ = jax.lax.while_loop(
        b_cond, b_body,
        (jnp.int32(0), jnp.zeros((rows, 1), jnp.int32),
         jnp.zeros((rows, 1), jnp.int32)))

    t = jax.lax.shift_left(p_pat - 32768, 16) | tl_pat

    tf = jax.lax.bitcast_convert_type(jnp.where(t < 0, t ^ mask, t), jnp.float32)

    e = jnp.exp(x - xmax)
    ge = keys >= t
    em = jnp.where(ge, e, 0.0)
    c_ge = jnp.sum(ge.astype(jnp.float32), axis=1, keepdims=True)
    sum_ge = jnp.sum(em, axis=1, keepdims=True)
    denom = sum_ge - (c_ge - jnp.float32(_K)) * jnp.exp(tf - xmax)
    o_ref[...] = em / denom


@functools.partial(jax.jit, static_argnums=())
def kernel(inputs):
    n_rows, depth = inputs.shape
    block_rows = 32
    grid = (n_rows // block_rows,)
    return pl.pallas_call(
        _body,
        grid=grid,
        in_specs=[pl.BlockSpec((block_rows, depth), lambda i: (i, 0))],
        out_specs=pl.BlockSpec((block_rows, depth), lambda i: (i, 0)),
        out_shape=jax.ShapeDtypeStruct((n_rows, depth), jnp.float32),
    )(inputs)
